# baseline (speedup 1.0000x reference)
import functools

import jax
import jax.numpy as jnp
from jax import lax
from jax.experimental import pallas as pl
from jax.experimental.pallas import tpu as pltpu
from jax.experimental.pallas import tpu_sc as plsc

_N = 10000
_NPAD = 10240
_NW = 32
_K = 128
_R = 2
_RI = 4
_RPW = _NPAD // 16
_RB = 2560

_sc_mesh = plsc.VectorSubcoreMesh(core_axis_name="c", subcore_axis_name="s")


@functools.partial(jax.jit, static_argnums=(4,))
def _segsum_sc(src_e, dst_e, table, zeros, F):
    n_e = src_e.shape[0]
    n_chunks = n_e // _K
    ch_max = -(-n_chunks // _NW)
    loop_hi = -(-ch_max // _RI) * _RI

    @functools.partial(
        pl.kernel,
        out_type=jax.ShapeDtypeStruct((2, _NPAD, F), jnp.float32),
        mesh=_sc_mesh,
        compiler_params=pltpu.CompilerParams(use_tc_tiling_on_sc=(F == 128)),
        scratch_types=[
            pltpu.VMEM((_RI, _K), jnp.int32),
            pltpu.VMEM((_RI, _K), jnp.int32),
            pltpu.VMEM((_R, _K, F), jnp.float32),
            pltpu.VMEM_SHARED((_NPAD, F), jnp.float32),
            pltpu.SemaphoreType.DMA,
        ] + [pltpu.SemaphoreType.DMA] * (_R + _RI),
    )
    def k(se_hbm, de_hbm, tab_hbm, z_hbm, out_hbm, sidx, didx, rows, acc,
          zsem, *sems):
        gsem = sems[:_R]
        isem = sems[_R:]
        c = lax.axis_index("c")
        s = lax.axis_index("s")
        wid = s * 2 + c
        nch = (n_chunks - wid + _NW - 1) // _NW

        def issue_idx(jj, q):
            base = (jj * _NW + wid) * _K
            pltpu.async_copy(se_hbm.at[pl.ds(base, _K)], sidx.at[q], isem[q])
            pltpu.async_copy(de_hbm.at[pl.ds(base, _K)], didx.at[q], isem[q])

        def wait_idx(q):
            pltpu.make_async_copy(se_hbm.at[pl.ds(0, _K)], sidx.at[q],
                                  isem[q]).wait()
            pltpu.make_async_copy(de_hbm.at[pl.ds(0, _K)], didx.at[q],
                                  isem[q]).wait()

        def issue_gather(q, b):
            pltpu.async_copy(tab_hbm.at[sidx.at[q]], rows.at[b], gsem[b])

        for q in range(_RI):
            issue_idx(q, q)
        for b in range(_R):
            wait_idx(b)
            issue_gather(b, b)

        pltpu.async_copy(z_hbm.at[pl.ds(s * _RPW, _RPW)],
                         acc.at[pl.ds(s * _RPW, _RPW)], zsem)
        pltpu.make_async_copy(z_hbm.at[pl.ds(0, _RPW)],
                              acc.at[pl.ds(0, _RPW)], zsem).wait()
        plsc.subcore_barrier()

        @pl.loop(0, loop_hi, step=_RI)
        def _(j):
            for b in range(_RI):
                jj = j + b
                buf = b % _R

                @pl.when(jj < nch)
                def _():
                    pltpu.make_async_copy(
                        tab_hbm.at[sidx.at[b]], rows.at[buf], gsem[buf]).wait()
                    pltpu.sync_copy(rows.at[buf], acc.at[didx.at[b]], add=True)

                @pl.when(jj + _RI < nch)
                def _():
                    issue_idx(jj + _RI, b)

                @pl.when(jj + _R < nch)
                def _():
                    wait_idx((b + _R) % _RI)
                    issue_gather((b + _R) % _RI, buf)

        plsc.subcore_barrier()
        pltpu.sync_copy(acc.at[pl.ds(s * _RPW, _RPW)],
                        out_hbm.at[c].at[pl.ds(s * _RPW, _RPW)])

    return k(src_e, dst_e, table, zeros)


@jax.jit
def _hist_sc(dst_e, zeros):
    n_e = dst_e.shape[0]
    n_chunks = n_e // _K
    loop_hi = -(-(-(-n_chunks // _NW)) // _RI) * _RI

    @functools.partial(
        pl.kernel,
        out_type=jax.ShapeDtypeStruct((2, _NPAD, 16), jnp.float32),
        mesh=_sc_mesh,
        compiler_params=pltpu.CompilerParams(use_tc_tiling_on_sc=False),
        scratch_types=[
            pltpu.VMEM((_RI, _K), jnp.int32),
            pltpu.VMEM((_K, 16), jnp.float32),
            pltpu.VMEM_SHARED((_NPAD, 16), jnp.float32),
            pltpu.SemaphoreType.DMA,
        ] + [pltpu.SemaphoreType.DMA] * _RI,
    )
    def k(de_hbm, z_hbm, out_hbm, didx, ones, acc, zsem, *isem):
        c = lax.axis_index("c")
        s = lax.axis_index("s")
        wid = s * 2 + c
        nch = (n_chunks - wid + _NW - 1) // _NW

        def issue_idx(jj, q):
            base = (jj * _NW + wid) * _K
            pltpu.async_copy(de_hbm.at[pl.ds(base, _K)], didx.at[q], isem[q])

        def wait_idx(q):
            pltpu.make_async_copy(de_hbm.at[pl.ds(0, _K)], didx.at[q],
                                  isem[q]).wait()

        for q in range(_RI):
            issue_idx(q, q)

        one = jnp.ones((16,), jnp.float32)

        @pl.loop(0, _K)
        def _(i):
            ones[i, pl.ds(0, 16)] = one

        pltpu.async_copy(z_hbm.at[pl.ds(s * _RPW, _RPW)],
                         acc.at[pl.ds(s * _RPW, _RPW)], zsem)
        pltpu.make_async_copy(z_hbm.at[pl.ds(0, _RPW)],
                              acc.at[pl.ds(0, _RPW)], zsem).wait()
        plsc.subcore_barrier()

        @pl.loop(0, loop_hi, step=_RI)
        def _(j):
            for b in range(_RI):
                jj = j + b

                @pl.when(jj < nch)
                def _():
                    wait_idx(b)
                    pltpu.sync_copy(ones, acc.at[didx.at[b]], add=True)

                @pl.when(jj + _RI < nch)
                def _():
                    issue_idx(jj + _RI, b)

        plsc.subcore_barrier()
        pltpu.sync_copy(acc.at[pl.ds(s * _RPW, _RPW)],
                        out_hbm.at[c].at[pl.ds(s * _RPW, _RPW)])

    return k(dst_e, zeros)


def _dinv_col(h_ref):
    deg = h_ref[0, :, 0:1] + h_ref[1, :, 0:1] + 1.0
    return lax.rsqrt(deg)


def _tc1_body(s1_ref, h_ref, w1_ref, b1_ref, wg_ref, o_ref):
    x = (s1_ref[0] + s1_ref[1]) * (1.0 / _N)
    h = jnp.maximum(jnp.dot(x, w1_ref[...],
                            preferred_element_type=jnp.float32) + b1_ref[...], 0.0)
    o_ref[...] = _dinv_col(h_ref) * jnp.dot(h, wg_ref[...],
                                            preferred_element_type=jnp.float32)


def _tc2_body(s2_ref, u1_ref, h_ref, b_ref, o_ref):
    agg = s2_ref[0] + s2_ref[1] + u1_ref[...]
    o_ref[...] = jnp.maximum(_dinv_col(h_ref) * agg + b_ref[...], 0.0)


def _tc3_body(s3_ref, h_ref, w2_ref, b2_ref, wg_ref, o_ref):
    x = (s3_ref[0] + s3_ref[1]) * (1.0 / _N)
    h = jnp.dot(x, w2_ref[...], preferred_element_type=jnp.float32) + b2_ref[...]
    o_ref[...] = _dinv_col(h_ref) * jnp.dot(h, wg_ref[...],
                                            preferred_element_type=jnp.float32)


def _tc4_body(s4_ref, u2_ref, h_ref, b_ref, o_ref):
    agg = s4_ref[0] + s4_ref[1] + u2_ref[...]
    z = _dinv_col(h_ref) * agg + b_ref[...]
    m = jnp.max(z, axis=1, keepdims=True)
    e = jnp.exp(z - m)
    lse = m + jnp.log(jnp.sum(e, axis=1, keepdims=True))
    o_ref[...] = z - lse


def _part3(cols):
    return pl.BlockSpec((2, _RB, cols), lambda i: (0, i, 0))


def _part2(cols, rb=_RB):
    return pl.BlockSpec((rb, cols), lambda i: (i, 0))


def _full(*shape):
    return pl.BlockSpec(shape, lambda i: (0,) * len(shape))


def _tc_call(fn, in_specs, out_spec, out_shape, *args, grid=4):
    return pl.pallas_call(
        fn,
        grid=(grid,),
        in_specs=in_specs,
        out_specs=out_spec,
        out_shape=jax.ShapeDtypeStruct(out_shape, jnp.float32),
    )(*args)


def kernel(x_in, edge_index_in, ego_edge_index,
           W_ego1, b_ego1, W_gcn1, b_gcn1,
           W_ego2, b_ego2, W_gcn2, b_gcn2):
    ei = edge_index_in.astype(jnp.int32)
    ee = ego_edge_index.astype(jnp.int32)
    zeros = jnp.zeros((_NPAD, 128), jnp.float32)
    zeros16 = jnp.zeros((_NPAD, 16), jnp.float32)
    zeros64 = jnp.zeros((_NPAD, 64), jnp.float32)

    ei0, ei1 = ei[0], ei[1]
    ee0, ee1 = ee[0], ee[1]
    hist = _hist_sc(ei1, zeros16)
    S1 = _segsum_sc(ee1, ee0, x_in, zeros, 128)

    u1 = _tc_call(
        _tc1_body,
        [_part3(128), _part3(16), _full(128, 128), _full(1, 128),
         _full(128, 128)],
        _part2(128), (_NPAD, 128),
        S1, hist, W_ego1, b_ego1.reshape(1, -1), W_gcn1)

    S2 = _segsum_sc(ei0, ei1, u1, zeros, 128)
    x2 = _tc_call(
        _tc2_body,
        [_part3(128), _part2(128), _part3(16), _full(1, 128)],
        _part2(128), (_NPAD, 128),
        S2, u1, hist, b_gcn1.reshape(1, -1))

    S3 = _segsum_sc(ee1, ee0, x2, zeros, 128)
    u2 = _tc_call(
        _tc3_body,
        [_part3(128), _part3(16), _full(128, 128), _full(1, 128),
         _full(128, 64)],
        _part2(64), (_NPAD, 64),
        S3, hist, W_ego2, b_ego2.reshape(1, -1), W_gcn2)

    S4 = _segsum_sc(ei0, ei1, u2, zeros64, 64)
    out = _tc_call(
        _tc4_body,
        [pl.BlockSpec((2, 1000, 64), lambda i: (0, i, 0)),
         _part2(64, 1000), pl.BlockSpec((2, 1000, 16), lambda i: (0, i, 0)),
         _full(1, 64)],
        _part2(64, 1000), (_N, 64),
        S4, u2, hist, b_gcn2.reshape(1, -1), grid=10)
    return out


# speedup vs baseline: 1.0238x; 1.0238x over previous
import functools

import jax
import jax.numpy as jnp
from jax import lax
from jax.experimental import pallas as pl
from jax.experimental.pallas import tpu as pltpu
from jax.experimental.pallas import tpu_sc as plsc

_N = 10000
_NPAD = 10240
_NW = 32
_K = 128
_R = 2
_RI = 4
_RPW = _NPAD // 16
_RB = 2560

_sc_mesh = plsc.VectorSubcoreMesh(core_axis_name="c", subcore_axis_name="s")


@functools.partial(jax.jit, static_argnums=(3, 4, 5))
def _segsum_sc(edges, table, zeros, srow, drow, F):
    n_e = edges.shape[1]
    n_chunks = n_e // _K
    ch_max = -(-n_chunks // _NW)
    loop_hi = -(-ch_max // _RI) * _RI

    @functools.partial(
        pl.kernel,
        out_type=jax.ShapeDtypeStruct((2, _NPAD, F), jnp.float32),
        mesh=_sc_mesh,
        compiler_params=pltpu.CompilerParams(use_tc_tiling_on_sc=(F == 128)),
        scratch_types=[
            pltpu.VMEM((_RI, _K), jnp.int32),
            pltpu.VMEM((_RI, _K), jnp.int32),
            pltpu.VMEM((_R, _K, F), jnp.float32),
            pltpu.VMEM_SHARED((_NPAD, F), jnp.float32),
            pltpu.SemaphoreType.DMA,
        ] + [pltpu.SemaphoreType.DMA] * (_R + _RI),
    )
    def k(e_hbm, tab_hbm, z_hbm, out_hbm, sidx, didx, rows, acc, zsem, *sems):
        gsem = sems[:_R]
        isem = sems[_R:]
        c = lax.axis_index("c")
        s = lax.axis_index("s")
        wid = s * 2 + c
        nch = (n_chunks - wid + _NW - 1) // _NW

        def issue_idx(jj, q):
            base = (jj * _NW + wid) * _K
            pltpu.async_copy(e_hbm.at[srow, pl.ds(base, _K)], sidx.at[q],
                             isem[q])
            pltpu.async_copy(e_hbm.at[drow, pl.ds(base, _K)], didx.at[q],
                             isem[q])

        def wait_idx(q):
            pltpu.make_async_copy(e_hbm.at[0, pl.ds(0, _K)], sidx.at[q],
                                  isem[q]).wait()
            pltpu.make_async_copy(e_hbm.at[0, pl.ds(0, _K)], didx.at[q],
                                  isem[q]).wait()

        def issue_gather(q, b):
            pltpu.async_copy(tab_hbm.at[sidx.at[q]], rows.at[b], gsem[b])

        for q in range(_RI):
            issue_idx(q, q)
        for b in range(_R):
            wait_idx(b)
            issue_gather(b, b)

        pltpu.async_copy(z_hbm.at[pl.ds(s * _RPW, _RPW)],
                         acc.at[pl.ds(s * _RPW, _RPW)], zsem)
        pltpu.make_async_copy(z_hbm.at[pl.ds(0, _RPW)],
                              acc.at[pl.ds(0, _RPW)], zsem).wait()
        plsc.subcore_barrier()

        @pl.loop(0, loop_hi, step=_RI)
        def _(j):
            for b in range(_RI):
                jj = j + b
                buf = b % _R

                @pl.when(jj < nch)
                def _():
                    pltpu.make_async_copy(
                        tab_hbm.at[sidx.at[b]], rows.at[buf], gsem[buf]).wait()
                    pltpu.sync_copy(rows.at[buf], acc.at[didx.at[b]], add=True)

                @pl.when(jj + _RI < nch)
                def _():
                    issue_idx(jj + _RI, b)

                @pl.when(jj + _R < nch)
                def _():
                    wait_idx((b + _R) % _RI)
                    issue_gather((b + _R) % _RI, buf)

        plsc.subcore_barrier()
        pltpu.sync_copy(acc.at[pl.ds(s * _RPW, _RPW)],
                        out_hbm.at[c].at[pl.ds(s * _RPW, _RPW)])

    return k(edges, table, zeros)


@functools.partial(jax.jit, static_argnums=(2,))
def _hist_sc(edges, zeros, drow):
    n_e = edges.shape[1]
    n_chunks = n_e // _K
    loop_hi = -(-(-(-n_chunks // _NW)) // _RI) * _RI

    @functools.partial(
        pl.kernel,
        out_type=jax.ShapeDtypeStruct((2, _NPAD, 16), jnp.float32),
        mesh=_sc_mesh,
        compiler_params=pltpu.CompilerParams(use_tc_tiling_on_sc=False),
        scratch_types=[
            pltpu.VMEM((_RI, _K), jnp.int32),
            pltpu.VMEM((_K, 16), jnp.float32),
            pltpu.VMEM_SHARED((_NPAD, 16), jnp.float32),
            pltpu.SemaphoreType.DMA,
        ] + [pltpu.SemaphoreType.DMA] * _RI,
    )
    def k(e_hbm, z_hbm, out_hbm, didx, ones, acc, zsem, *isem):
        c = lax.axis_index("c")
        s = lax.axis_index("s")
        wid = s * 2 + c
        nch = (n_chunks - wid + _NW - 1) // _NW

        def issue_idx(jj, q):
            base = (jj * _NW + wid) * _K
            pltpu.async_copy(e_hbm.at[drow, pl.ds(base, _K)], didx.at[q],
                             isem[q])

        def wait_idx(q):
            pltpu.make_async_copy(e_hbm.at[0, pl.ds(0, _K)], didx.at[q],
                                  isem[q]).wait()

        for q in range(_RI):
            issue_idx(q, q)

        one = jnp.ones((16,), jnp.float32)

        @pl.loop(0, _K)
        def _(i):
            ones[i, pl.ds(0, 16)] = one

        pltpu.async_copy(z_hbm.at[pl.ds(s * _RPW, _RPW)],
                         acc.at[pl.ds(s * _RPW, _RPW)], zsem)
        pltpu.make_async_copy(z_hbm.at[pl.ds(0, _RPW)],
                              acc.at[pl.ds(0, _RPW)], zsem).wait()
        plsc.subcore_barrier()

        @pl.loop(0, loop_hi, step=_RI)
        def _(j):
            for b in range(_RI):
                jj = j + b

                @pl.when(jj < nch)
                def _():
                    wait_idx(b)
                    pltpu.sync_copy(ones, acc.at[didx.at[b]], add=True)

                @pl.when(jj + _RI < nch)
                def _():
                    issue_idx(jj + _RI, b)

        plsc.subcore_barrier()
        pltpu.sync_copy(acc.at[pl.ds(s * _RPW, _RPW)],
                        out_hbm.at[c].at[pl.ds(s * _RPW, _RPW)])

    return k(edges, zeros)


def _dinv_col(h_ref):
    deg = h_ref[0, :, 0:1] + h_ref[1, :, 0:1] + 1.0
    return lax.rsqrt(deg)


def _tc1_body(s1_ref, h_ref, w1_ref, b1_ref, wg_ref, o_ref):
    x = (s1_ref[0] + s1_ref[1]) * (1.0 / _N)
    h = jnp.maximum(jnp.dot(x, w1_ref[...],
                            preferred_element_type=jnp.float32) + b1_ref[...], 0.0)
    o_ref[...] = _dinv_col(h_ref) * jnp.dot(h, wg_ref[...],
                                            preferred_element_type=jnp.float32)


def _tc2_body(s2_ref, u1_ref, h_ref, b_ref, o_ref):
    agg = s2_ref[0] + s2_ref[1] + u1_ref[...]
    o_ref[...] = jnp.maximum(_dinv_col(h_ref) * agg + b_ref[...], 0.0)


def _tc3_body(s3_ref, h_ref, w2_ref, b2_ref, wg_ref, o_ref):
    x = (s3_ref[0] + s3_ref[1]) * (1.0 / _N)
    h = jnp.dot(x, w2_ref[...], preferred_element_type=jnp.float32) + b2_ref[...]
    o_ref[...] = _dinv_col(h_ref) * jnp.dot(h, wg_ref[...],
                                            preferred_element_type=jnp.float32)


def _tc4_body(s4_ref, u2_ref, h_ref, b_ref, o_ref):
    agg = s4_ref[0] + s4_ref[1] + u2_ref[...]
    z = _dinv_col(h_ref) * agg + b_ref[...]
    m = jnp.max(z, axis=1, keepdims=True)
    e = jnp.exp(z - m)
    lse = m + jnp.log(jnp.sum(e, axis=1, keepdims=True))
    o_ref[...] = z - lse


def _part3(cols):
    return pl.BlockSpec((2, _RB, cols), lambda i: (0, i, 0))


def _part2(cols, rb=_RB):
    return pl.BlockSpec((rb, cols), lambda i: (i, 0))


def _full(*shape):
    return pl.BlockSpec(shape, lambda i: (0,) * len(shape))


def _tc_call(fn, in_specs, out_spec, out_shape, *args, grid=4):
    return pl.pallas_call(
        fn,
        grid=(grid,),
        in_specs=in_specs,
        out_specs=out_spec,
        out_shape=jax.ShapeDtypeStruct(out_shape, jnp.float32),
    )(*args)


def kernel(x_in, edge_index_in, ego_edge_index,
           W_ego1, b_ego1, W_gcn1, b_gcn1,
           W_ego2, b_ego2, W_gcn2, b_gcn2):
    ei = edge_index_in.astype(jnp.int32)
    ee = ego_edge_index.astype(jnp.int32)
    zeros = jnp.zeros((_NPAD, 128), jnp.float32)
    zeros16 = jnp.zeros((_NPAD, 16), jnp.float32)
    zeros64 = jnp.zeros((_NPAD, 64), jnp.float32)

    hist = _hist_sc(ei, zeros16, 1)
    S1 = _segsum_sc(ee, x_in, zeros, 1, 0, 128)

    u1 = _tc_call(
        _tc1_body,
        [_part3(128), _part3(16), _full(128, 128), _full(1, 128),
         _full(128, 128)],
        _part2(128), (_NPAD, 128),
        S1, hist, W_ego1, b_ego1.reshape(1, -1), W_gcn1)

    S2 = _segsum_sc(ei, u1, zeros, 0, 1, 128)
    x2 = _tc_call(
        _tc2_body,
        [_part3(128), _part2(128), _part3(16), _full(1, 128)],
        _part2(128), (_NPAD, 128),
        S2, u1, hist, b_gcn1.reshape(1, -1))

    S3 = _segsum_sc(ee, x2, zeros, 1, 0, 128)
    u2 = _tc_call(
        _tc3_body,
        [_part3(128), _part3(16), _full(128, 128), _full(1, 128),
         _full(128, 64)],
        _part2(64), (_NPAD, 64),
        S3, hist, W_ego2, b_ego2.reshape(1, -1), W_gcn2)

    S4 = _segsum_sc(ei, u2, zeros64, 0, 1, 64)
    out = _tc_call(
        _tc4_body,
        [pl.BlockSpec((2, 1000, 64), lambda i: (0, i, 0)),
         _part2(64, 1000), pl.BlockSpec((2, 1000, 16), lambda i: (0, i, 0)),
         _full(1, 64)],
        _part2(64, 1000), (_N, 64),
        S4, u2, hist, b_gcn2.reshape(1, -1), grid=10)
    return out
